# baseline (speedup 1.0000x reference)
import functools

import jax
import jax.numpy as jnp
from jax import lax
from jax.experimental import pallas as pl
from jax.experimental.pallas import tpu as pltpu
from jax.experimental.pallas import tpu_sc as plsc

N = 128 * 32768
NC, NS, L = 2, 16, 16
NW = NC * NS
PER_W = N // NW
CHUNK = 32768
NCH = PER_W // CHUNK
UNROLL = 4
STEPS = CHUNK // (L * UNROLL)

_I32_MAX = jnp.iinfo(jnp.int32).max


def _scan_chunk(buf, carry):
    def body(i, cr):
        vcnt = cr[0]
        vmaxs = list(cr[1:1 + UNROLL])
        vsteps = list(cr[1 + UNROLL:1 + 2 * UNROLL])
        for k in range(UNROLL):
            x = buf[pl.ds(i * (L * UNROLL) + k * L, L)]
            m = x > vmaxs[k]
            vmaxs[k] = jnp.where(m, x, vmaxs[k])
            vsteps[k] = jnp.where(m, vcnt, vsteps[k])
        return (vcnt + UNROLL, *vmaxs, *vsteps)

    return lax.fori_loop(0, STEPS, body, carry, unroll=False)


def _stage1_body(x_hbm, outv_hbm, outi_hbm, buf0, buf1, ov, oi, sem0, sem1):
    c = lax.axis_index("c")
    s = lax.axis_index("s")
    wid = s * NC + c
    base = wid * PER_W

    bufs = (buf0, buf1)
    sems = (sem0, sem1)

    neg_inf = jnp.full((L,), -jnp.inf, jnp.float32)
    zero = jnp.zeros((L,), jnp.int32)
    carry = (zero,) + (neg_inf,) * UNROLL + (zero,) * UNROLL

    copies = [None, None]
    copies[0] = pltpu.async_copy(x_hbm.at[pl.ds(base, CHUNK)], buf0, sem0)
    for ci in range(NCH):
        cur = ci % 2
        nxt = (ci + 1) % 2
        if ci + 1 < NCH:
            copies[nxt] = pltpu.async_copy(
                x_hbm.at[pl.ds(base + (ci + 1) * CHUNK, CHUNK)],
                bufs[nxt], sems[nxt])
        copies[cur].wait()
        carry = _scan_chunk(bufs[cur], carry)

    vmaxs = carry[1:1 + UNROLL]
    vsteps = carry[1 + UNROLL:1 + 2 * UNROLL]
    lane = lax.broadcasted_iota(jnp.int32, (L,), 0)

    vm = vmaxs[0]
    vi = (vsteps[0] + 0) * L + lane + base
    for k in range(1, UNROLL):
        mb = vmaxs[k]
        ib = (vsteps[k] + k) * L + lane + base
        take = (mb > vm) | ((mb == vm) & (ib < vi))
        vm = jnp.where(take, mb, vm)
        vi = jnp.where(take, ib, vi)

    ov[...] = vm
    oi[...] = vi
    pltpu.sync_copy(ov, outv_hbm.at[wid])
    pltpu.sync_copy(oi, outi_hbm.at[wid])


@functools.partial(
    pl.kernel,
    out_type=(
        jax.ShapeDtypeStruct((NW, L), jnp.float32),
        jax.ShapeDtypeStruct((NW, L), jnp.int32),
    ),
    mesh=plsc.VectorSubcoreMesh(core_axis_name="c", subcore_axis_name="s"),
    scratch_types=(
        pltpu.VMEM((CHUNK,), jnp.float32),
        pltpu.VMEM((CHUNK,), jnp.float32),
        pltpu.VMEM((L,), jnp.float32),
        pltpu.VMEM((L,), jnp.int32),
        pltpu.SemaphoreType.DMA,
        pltpu.SemaphoreType.DMA,
    ),
)
def _stage1(x_hbm, outv_hbm, outi_hbm, buf0, buf1, ov, oi, sem0, sem1):
    _stage1_body(x_hbm, outv_hbm, outi_hbm, buf0, buf1, ov, oi, sem0, sem1)


def _merge_body(v_ref, i_ref, o_ref):
    v = v_ref[...]
    idx = i_ref[...]
    m = jnp.max(v)
    sel = jnp.where(v == m, idx, _I32_MAX)
    o_ref[0, 0] = jnp.min(sel)


_merge = pl.pallas_call(
    _merge_body,
    out_shape=jax.ShapeDtypeStruct((1, 1), jnp.int32),
    out_specs=pl.BlockSpec(memory_space=pltpu.SMEM),
)


@jax.jit
def kernel(input):
    x = input.reshape(-1)
    pv, pi = _stage1(x)
    return _merge(pv.reshape(4, 128), pi.reshape(4, 128))


# speedup vs baseline: 1.6733x; 1.6733x over previous
import functools

import jax
import jax.numpy as jnp
from jax import lax
from jax.experimental import pallas as pl
from jax.experimental.pallas import tpu as pltpu
from jax.experimental.pallas import tpu_sc as plsc

R, C = 128, 32768
NC, NS, L = 2, 16, 16
NW = NC * NS
RPB = 8
HALF = C // 2
CW = 4096
NCH = HALF // CW
VSTEPS = CW // L

_I32_MAX = jnp.iinfo(jnp.int32).max


def _scan_chunk(buf, carry):
    def body(j, cr):
        vcnt = cr[0]
        vmaxs = list(cr[1:1 + RPB])
        vsteps = list(cr[1 + RPB:1 + 2 * RPB])
        for r in range(RPB):
            x = buf[r, pl.ds(j * L, L)]
            m = x > vmaxs[r]
            vmaxs[r] = jnp.where(m, x, vmaxs[r])
            vsteps[r] = jnp.where(m, vcnt, vsteps[r])
        return (vcnt + 1, *vmaxs, *vsteps)

    return lax.fori_loop(0, VSTEPS, body, carry, unroll=False)


@functools.partial(
    pl.kernel,
    out_type=(
        jax.ShapeDtypeStruct((NW * L,), jnp.float32),
        jax.ShapeDtypeStruct((NW * L,), jnp.int32),
    ),
    mesh=plsc.VectorSubcoreMesh(core_axis_name="c", subcore_axis_name="s"),
    scratch_types=(
        pltpu.VMEM((RPB, CW), jnp.float32),
        pltpu.VMEM((RPB, CW), jnp.float32),
        pltpu.VMEM((L,), jnp.float32),
        pltpu.VMEM((L,), jnp.int32),
        pltpu.SemaphoreType.DMA,
        pltpu.SemaphoreType.DMA,
    ),
    compiler_params=pltpu.CompilerParams(use_tc_tiling_on_sc=True),
)
def _stage1(x_hbm, outv_hbm, outi_hbm, buf0, buf1, ov, oi, sem0, sem1):
    c = lax.axis_index("c")
    s = lax.axis_index("s")
    wid = s * NC + c
    row0 = (wid // 2) * RPB
    c0 = (wid % 2) * HALF

    bufs = (buf0, buf1)
    sems = (sem0, sem1)

    neg_inf = jnp.full((L,), -jnp.inf, jnp.float32)
    zero = jnp.zeros((L,), jnp.int32)
    carry = (zero,) + (neg_inf,) * RPB + (zero,) * RPB

    copies = [None, None]
    copies[0] = pltpu.async_copy(
        x_hbm.at[pl.ds(row0, RPB), pl.ds(c0, CW)], buf0, sem0)
    for ci in range(NCH):
        cur = ci % 2
        nxt = (ci + 1) % 2
        if ci + 1 < NCH:
            copies[nxt] = pltpu.async_copy(
                x_hbm.at[pl.ds(row0, RPB), pl.ds(c0 + (ci + 1) * CW, CW)],
                bufs[nxt], sems[nxt])
        copies[cur].wait()
        carry = _scan_chunk(bufs[cur], carry)

    vmaxs = carry[1:1 + RPB]
    vsteps = carry[1 + RPB:1 + 2 * RPB]
    lane = lax.broadcasted_iota(jnp.int32, (L,), 0)

    vm = vmaxs[0]
    vi = (row0 + 0) * C + c0 + vsteps[0] * L + lane
    for r in range(1, RPB):
        mb = vmaxs[r]
        ib = (row0 + r) * C + c0 + vsteps[r] * L + lane
        take = (mb > vm) | ((mb == vm) & (ib < vi))
        vm = jnp.where(take, mb, vm)
        vi = jnp.where(take, ib, vi)

    ov[...] = vm
    oi[...] = vi
    pltpu.sync_copy(ov, outv_hbm.at[pl.ds(wid * L, L)])
    pltpu.sync_copy(oi, outi_hbm.at[pl.ds(wid * L, L)])


def _merge_body(v_ref, i_ref, o_ref):
    v = v_ref[...]
    idx = i_ref[...]
    m = jnp.max(v)
    sel = jnp.where(v == m, idx, _I32_MAX)
    o_ref[0, 0] = jnp.min(sel)


_merge = pl.pallas_call(
    _merge_body,
    out_shape=jax.ShapeDtypeStruct((1, 1), jnp.int32),
    out_specs=pl.BlockSpec(memory_space=pltpu.SMEM),
)


@jax.jit
def kernel(input):
    pv, pi = _stage1(input)
    return _merge(pv.reshape(4, 128), pi.reshape(4, 128))


# speedup vs baseline: 1.7544x; 1.0484x over previous
import functools

import jax
import jax.numpy as jnp
from jax import lax
from jax.experimental import pallas as pl
from jax.experimental.pallas import tpu as pltpu
from jax.experimental.pallas import tpu_sc as plsc

R, C = 128, 32768
NC, NS, L = 2, 16, 16
NW = NC * NS
RPB = 8
SC_ROWS = 64
QC = 8192
CW = 4096
NCH = QC // CW
VSTEPS = CW // L

TC_R0 = SC_ROWS
TC_ROWS = R - SC_ROWS
TCB = 4096
TC_G = C // TCB

_I32_MAX = jnp.iinfo(jnp.int32).max


def _scan_chunk(buf, carry):
    def body(j, cr):
        vcnt = cr[0]
        vmaxs = list(cr[1:1 + RPB])
        vsteps = list(cr[1 + RPB:1 + 2 * RPB])
        for r in range(RPB):
            x = buf[r, pl.ds(j * L, L)]
            m = x > vmaxs[r]
            vmaxs[r] = jnp.where(m, x, vmaxs[r])
            vsteps[r] = jnp.where(m, vcnt, vsteps[r])
        return (vcnt + 1, *vmaxs, *vsteps)

    return lax.fori_loop(0, VSTEPS, body, carry, unroll=2)


@functools.partial(
    pl.kernel,
    out_type=(
        jax.ShapeDtypeStruct((NW * L,), jnp.float32),
        jax.ShapeDtypeStruct((NW * L,), jnp.int32),
    ),
    mesh=plsc.VectorSubcoreMesh(core_axis_name="c", subcore_axis_name="s"),
    scratch_types=(
        pltpu.VMEM((RPB, CW), jnp.float32),
        pltpu.VMEM((RPB, CW), jnp.float32),
        pltpu.VMEM((L,), jnp.float32),
        pltpu.VMEM((L,), jnp.int32),
        pltpu.SemaphoreType.DMA,
        pltpu.SemaphoreType.DMA,
    ),
    compiler_params=pltpu.CompilerParams(use_tc_tiling_on_sc=True),
)
def _stage1_sc(x_hbm, outv_hbm, outi_hbm, buf0, buf1, ov, oi, sem0, sem1):
    c = lax.axis_index("c")
    s = lax.axis_index("s")
    wid = s * NC + c
    row0 = (wid // 4) * RPB
    c0 = (wid % 4) * QC

    bufs = (buf0, buf1)
    sems = (sem0, sem1)

    neg_inf = jnp.full((L,), -jnp.inf, jnp.float32)
    zero = jnp.zeros((L,), jnp.int32)
    carry = (zero,) + (neg_inf,) * RPB + (zero,) * RPB

    copies = [None, None]
    copies[0] = pltpu.async_copy(
        x_hbm.at[pl.ds(row0, RPB), pl.ds(c0, CW)], buf0, sem0)
    for ci in range(NCH):
        cur = ci % 2
        nxt = (ci + 1) % 2
        if ci + 1 < NCH:
            copies[nxt] = pltpu.async_copy(
                x_hbm.at[pl.ds(row0, RPB), pl.ds(c0 + (ci + 1) * CW, CW)],
                bufs[nxt], sems[nxt])
        copies[cur].wait()
        carry = _scan_chunk(bufs[cur], carry)

    vmaxs = carry[1:1 + RPB]
    vsteps = carry[1 + RPB:1 + 2 * RPB]
    lane = lax.broadcasted_iota(jnp.int32, (L,), 0)

    vm = vmaxs[0]
    vi = (row0 + 0) * C + c0 + vsteps[0] * L + lane
    for r in range(1, RPB):
        mb = vmaxs[r]
        ib = (row0 + r) * C + c0 + vsteps[r] * L + lane
        take = (mb > vm) | ((mb == vm) & (ib < vi))
        vm = jnp.where(take, mb, vm)
        vi = jnp.where(take, ib, vi)

    ov[...] = vm
    oi[...] = vi
    pltpu.sync_copy(ov, outv_hbm.at[pl.ds(wid * L, L)])
    pltpu.sync_copy(oi, outi_hbm.at[pl.ds(wid * L, L)])


def _stage1_tc_body(x_ref, om_ref, oi_ref, sm, si):
    g = pl.program_id(0)
    x = x_ref[...]
    m = jnp.max(x)
    rows = lax.broadcasted_iota(jnp.int32, (TC_ROWS, TCB), 0)
    cols = lax.broadcasted_iota(jnp.int32, (TC_ROWS, TCB), 1)
    idx_local = rows * C + cols
    li = jnp.min(jnp.where(x == m, idx_local, _I32_MAX))
    li = li + (TC_R0 * C + g * TCB)

    @pl.when(g == 0)
    def _():
        sm[0] = m
        si[0] = li

    @pl.when(g > 0)
    def _():
        better = (m > sm[0]) | ((m == sm[0]) & (li < si[0]))

        @pl.when(better)
        def _():
            sm[0] = m
            si[0] = li

    @pl.when(g == TC_G - 1)
    def _():
        om_ref[0, 0] = sm[0]
        oi_ref[0, 0] = si[0]


_stage1_tc = pl.pallas_call(
    _stage1_tc_body,
    grid=(TC_G,),
    in_specs=[pl.BlockSpec((TC_ROWS, TCB), lambda g: (1, g))],
    out_specs=(
        pl.BlockSpec(memory_space=pltpu.SMEM),
        pl.BlockSpec(memory_space=pltpu.SMEM),
    ),
    out_shape=(
        jax.ShapeDtypeStruct((1, 1), jnp.float32),
        jax.ShapeDtypeStruct((1, 1), jnp.int32),
    ),
    scratch_shapes=[
        pltpu.SMEM((1,), jnp.float32),
        pltpu.SMEM((1,), jnp.int32),
    ],
)


def _merge_body(v_ref, i_ref, tm_ref, ti_ref, o_ref):
    v = v_ref[...]
    idx = i_ref[...]
    tm = tm_ref[0, 0]
    ti = ti_ref[0, 0]
    m = jnp.maximum(jnp.max(v), tm)
    best_sc = jnp.min(jnp.where(v == m, idx, _I32_MAX))
    o_ref[0, 0] = jnp.where(tm == m, jnp.minimum(best_sc, ti), best_sc)


_merge = pl.pallas_call(
    _merge_body,
    in_specs=[
        pl.BlockSpec(memory_space=pltpu.VMEM),
        pl.BlockSpec(memory_space=pltpu.VMEM),
        pl.BlockSpec(memory_space=pltpu.SMEM),
        pl.BlockSpec(memory_space=pltpu.SMEM),
    ],
    out_shape=jax.ShapeDtypeStruct((1, 1), jnp.int32),
    out_specs=pl.BlockSpec(memory_space=pltpu.SMEM),
)


@jax.jit
def kernel(input):
    pv, pi = _stage1_sc(input)
    tm, ti = _stage1_tc(input)
    return _merge(pv.reshape(4, 128), pi.reshape(4, 128), tm, ti)
